# baseline (speedup 1.0000x reference)
import functools

import jax
import jax.numpy as jnp
from jax.experimental import pallas as pl
from jax.experimental.pallas import tpu as pltpu

B = 128
N = 100000

K0 = 0
K1 = 42
K2 = K0 ^ K1 ^ 0x1BD11BDA

_ROT_A = (13, 15, 26, 6)
_ROT_B = (17, 29, 16, 24)
_TINY = 1.1754943508222875e-38


def _threefry_bits(x1):
    ks = (jnp.uint32(K0), jnp.uint32(K1), jnp.uint32(K2))
    x0 = jnp.zeros_like(x1)
    rots = (_ROT_A, _ROT_B, _ROT_A, _ROT_B, _ROT_A)
    inj = ((1, 2), (2, 0), (0, 1), (1, 2), (2, 0))
    for i in range(5):
        for r in rots[i]:
            x0 = x0 + x1
            x1 = (x1 << jnp.uint32(r)) | (x1 >> jnp.uint32(32 - r))
            x1 = x1 ^ x0
        a, b = inj[i]
        x0 = x0 + ks[a]
        x1 = x1 + ks[b] + jnp.uint32(i + 1)
    return x0 ^ x1


def _sample_kernel(logits_ref, out_ref, max_ref, idx_ref, *, block_b, block_n,
                   n_col_blocks):
    i = pl.program_id(0)
    j = pl.program_id(1)

    r0 = i * block_b
    c0 = j * block_n

    row = jax.lax.broadcasted_iota(jnp.int32, (block_b, block_n), 0)
    col = jax.lax.broadcasted_iota(jnp.int32, (block_b, block_n), 1)
    ctr = ((r0 + row) * N + (c0 + col) + K1).astype(jnp.uint32)

    bits = _threefry_bits(ctr)

    f = jax.lax.bitcast_convert_type(
        (bits >> jnp.uint32(9)) | jnp.uint32(0x3F800000), jnp.float32
    ) - jnp.float32(1.0)
    u = jnp.maximum(
        jnp.float32(_TINY), f * (jnp.float32(1.0) - jnp.float32(_TINY)) + jnp.float32(_TINY)
    )
    g = -jnp.log(-jnp.log(u))

    w = g + logits_ref[...]

    col_global = c0 + col
    w = jnp.where(col_global < N, w, -jnp.inf)

    m = jnp.max(w, axis=1, keepdims=True)
    hit = w == m
    tidx = jnp.min(
        jnp.where(hit, col_global, jnp.int32(0x7FFFFFFF)), axis=1, keepdims=True
    )

    @pl.when(j == 0)
    def _():
        max_ref[...] = jnp.full_like(max_ref, -jnp.inf)
        idx_ref[...] = jnp.zeros_like(idx_ref)

    better = m > max_ref[...]
    idx_ref[...] = jnp.where(better, tidx, idx_ref[...])
    max_ref[...] = jnp.where(better, m, max_ref[...])

    @pl.when(j == n_col_blocks - 1)
    def _():
        out_ref[...] = idx_ref[...]


@jax.jit
def kernel(logits):
    block_b = 64
    block_n = 2048
    n_row_blocks = B // block_b
    n_col_blocks = pl.cdiv(N, block_n)

    out = pl.pallas_call(
        functools.partial(
            _sample_kernel, block_b=block_b, block_n=block_n,
            n_col_blocks=n_col_blocks,
        ),
        grid=(n_row_blocks, n_col_blocks),
        in_specs=[pl.BlockSpec((block_b, block_n), lambda i, j: (i, j))],
        out_specs=pl.BlockSpec((block_b, 1), lambda i, j: (i, 0)),
        out_shape=jax.ShapeDtypeStruct((B, 1), jnp.int32),
        scratch_shapes=[
            pltpu.VMEM((block_b, 1), jnp.float32),
            pltpu.VMEM((block_b, 1), jnp.int32),
        ],
        compiler_params=pltpu.CompilerParams(
            dimension_semantics=("parallel", "arbitrary"),
        ),
    )(logits)
    return out.reshape(B)
